# baseline (speedup 1.0000x reference)
import functools

import jax
import jax.numpy as jnp
from jax import lax
from jax.experimental import pallas as pl
from jax.experimental.pallas import tpu as pltpu
from jax.experimental.pallas import tpu_sc as plsc

N = 10000
E = 320000
NE = E + N
NC, NS, L = 2, 16, 16
W = NC * NS
BLK = 96
K = 6
NBLK0 = 96
NBLK1 = 120
EP = (NBLK0 + NBLK1) * NS * BLK
NACC = 10240
TROWS = NACC // NS

_f32 = jnp.float32
_i32 = jnp.int32


def _t1_body(x_ref, w1_ref, asrc_ref, adst_ref, table_ref, adstv_ref):
    h = jnp.dot(x_ref[...], w1_ref[...], preferred_element_type=_f32)
    table_ref[:, 0:64] = h
    table_ref[:, 64:72] = jnp.dot(h, asrc_ref[...], preferred_element_type=_f32)
    table_ref[:, 72:80] = jnp.ones((h.shape[0], 8), _f32)
    adstv_ref[...] = jnp.dot(h, adst_ref[...], preferred_element_type=_f32)


def _t2_body(p0_ref, p1_ref, b1_ref, w2_ref, rmap_ref, as2_ref, ad2_ref,
             table_ref, adstv_ref):
    acc = p0_ref[...] + p1_ref[...]
    recip = 1.0 / acc[:, 72:80]
    out1 = acc[:, 0:64] * jnp.dot(recip, rmap_ref[...],
                                  preferred_element_type=_f32) + b1_ref[...]
    hact = jnp.where(out1 > 0, out1, jnp.exp(out1) - 1.0)
    h2 = jnp.dot(hact, w2_ref[...], preferred_element_type=_f32)
    table_ref[:, 0:128] = h2
    table_ref[:, 128:129] = jnp.dot(h2, as2_ref[...], preferred_element_type=_f32)
    table_ref[:, 129:130] = jnp.ones((h2.shape[0], 1), _f32)
    table_ref[:, 130:144] = jnp.zeros((h2.shape[0], 14), _f32)
    adstv_ref[...] = jnp.dot(h2, ad2_ref[...], preferred_element_type=_f32)


def _t3_body(p0_ref, p1_ref, b2_ref, out_ref):
    acc = p0_ref[...] + p1_ref[...]
    out_ref[...] = acc[:, 0:128] / acc[:, 129:130] + b2_ref[...]


def _tc_call(body, ins, out_shapes, in_blocks, out_blocks, grid):
    return pl.pallas_call(
        body,
        grid=grid,
        in_specs=in_blocks,
        out_specs=out_blocks,
        out_shape=out_shapes,
    )(*ins)


def _sc_edge_kernel(table, adst, src_p, dst_p, out, acc,
                    sidx0, sidx1, didx0, didx1, rows0, rows1, adb0, adb1,
                    alpha1, gsemT0, gsemT1, gsemA0, gsemA1, ssem0, ssem1,
                    isem0, isem1, *, ncols, nvec, layer1):
    c = lax.axis_index("c")
    s = lax.axis_index("s")
    iota = lax.iota(_i32, L)
    sidx = (sidx0, sidx1)
    didx = (didx0, didx1)
    rows = (rows0, rows1)
    adb = (adb0, adb1)
    gsemT = (gsemT0, gsemT1)
    gsemA = (gsemA0, gsemA1)
    ssem = (ssem0, ssem1)
    isem = (isem0, isem1)

    zero = jnp.zeros((L,), _f32)

    def _zr(r, carry):
        for j in range(nvec):
            rows0[r, pl.ds(16 * j, 16)] = zero
        return carry

    lax.fori_loop(0, BLK, _zr, 0)
    tb = s * TROWS
    nz = TROWS // BLK
    for z in range(nz):
        pltpu.sync_copy(rows0, acc.at[pl.ds(tb + z * BLK, BLK)])
    rem = TROWS - nz * BLK
    if rem:
        pltpu.sync_copy(rows0.at[pl.ds(0, rem)],
                        acc.at[pl.ds(tb + nz * BLK, rem)])
    plsc.subcore_barrier()

    sbrow = jnp.where(c == 0, s * NBLK0, NS * NBLK0 + s * NBLK1)
    nsb_half = jnp.where(c == 0, NBLK0 // K // 2, NBLK1 // K // 2)
    if layer1:
        colpats = [2 * j + lax.shift_right_logical(iota, 3) for j in range(4)]
        colpats.append(iota & 7)

    def _sb_load(sb, q, sem_q):
        pltpu.async_copy(src_p.at[pl.ds(sbrow + sb * K, K)], sidx[q], sem_q)
        pltpu.async_copy(dst_p.at[pl.ds(sbrow + sb * K, K)], didx[q], sem_q)

    def _sb_wait(q):
        pltpu.make_async_copy(src_p.at[pl.ds(0, K)], sidx[q], isem[q]).wait()
        pltpu.make_async_copy(dst_p.at[pl.ds(0, K)], didx[q], isem[q]).wait()

    def _gather_issue(q, k, p):
        pltpu.async_copy(table.at[sidx[q].at[k]], rows[p], gsemT[p])
        pltpu.async_copy(adst.at[didx[q].at[k]], adb[p], gsemA[p])

    def _gather_wait(p):
        pltpu.make_async_copy(table.at[sidx[0].at[0]], rows[p], gsemT[p]).wait()
        pltpu.make_async_copy(adst.at[didx[0].at[0]], adb[p], gsemA[p]).wait()

    def _scatter_issue(q, k, p):
        pltpu.async_copy(rows[p], acc.at[didx[q].at[k]], ssem[p], add=True)

    def _scatter_wait(p):
        pltpu.make_async_copy(rows[p], acc.at[didx[0].at[0]], ssem[p]).wait()

    def _compute(p):
        rp, ap = rows[p], adb[p]

        @plsc.parallel_loop(0, BLK, unroll=4)
        def _edge(e):
            a_s = rp[e, pl.ds(64 if layer1 else 128, 16)]
            a_d = ap[e, pl.ds(0, 16)]
            ssum = a_s + a_d
            alpha = jnp.exp(jnp.where(ssum > 0, ssum, 0.2 * ssum))
            alpha1[pl.ds(e * 16, 16)] = alpha
            if layer1:
                for j in range(nvec - 1):
                    av = plsc.load_gather(alpha1, [e * 16 + colpats[j]])
                    rp[e, pl.ds(16 * j, 16)] = rp[e, pl.ds(16 * j, 16)] * av
                av4 = plsc.load_gather(alpha1, [e * 16 + colpats[nvec - 1]])
                rp[e, pl.ds(16 * (nvec - 1), 16)] = av4
            else:
                av = plsc.load_gather(alpha1, [jnp.full((L,), e * 16, _i32)])
                for j in range(nvec - 1):
                    rp[e, pl.ds(16 * j, 16)] = rp[e, pl.ds(16 * j, 16)] * av
                rp[e, pl.ds(16 * (nvec - 1), 16)] = av

    _sb_load(0, 0, isem[0])
    _sb_wait(0)
    _sb_load(1, 1, isem[1])
    _gather_issue(0, 0, 0)

    def _t_step(t, carry):
        for qq in (0, 1):
            sb = 2 * t + qq
            for k in range(K):
                p = k % 2
                _gather_wait(p)
                if k == 0:
                    if qq == 0:
                        @pl.when(t > 0)
                        def _():
                            _scatter_wait(1 - p)
                            _sb_load(sb + 1, 1 - qq, isem[1 - qq])
                    else:
                        _scatter_wait(1 - p)

                        @pl.when(t < nsb_half - 1)
                        def _():
                            _sb_load(sb + 1, 1 - qq, isem[1 - qq])
                else:
                    _scatter_wait(1 - p)
                if k < K - 1:
                    _gather_issue(qq, k + 1, 1 - p)
                else:
                    if qq == 0:
                        _sb_wait(1 - qq)
                        _gather_issue(1 - qq, 0, 1 - p)
                    else:
                        @pl.when(t < nsb_half - 1)
                        def _():
                            _sb_wait(1 - qq)
                            _gather_issue(1 - qq, 0, 1 - p)
                _compute(p)
                _scatter_issue(qq, k, p)
        return carry

    lax.fori_loop(0, nsb_half, _t_step, 0)
    _scatter_wait(1)
    plsc.subcore_barrier()

    pltpu.sync_copy(acc.at[pl.ds(tb, TROWS)], out.at[c, pl.ds(tb, TROWS)])


def _make_sc_layer(ncols, layer1):
    nvec = ncols // 16
    mesh = plsc.VectorSubcoreMesh(core_axis_name="c", subcore_axis_name="s")
    return pl.kernel(
        functools.partial(_sc_edge_kernel, ncols=ncols, nvec=nvec,
                          layer1=layer1),
        out_type=jax.ShapeDtypeStruct((NC, NACC, ncols), _f32),
        mesh=mesh,
        compiler_params=pltpu.CompilerParams(needs_layout_passes=False,
                                             use_tc_tiling_on_sc=False),
        scratch_types=[
            pltpu.VMEM_SHARED((NACC, ncols), _f32),
            pltpu.VMEM((K, BLK), _i32),
            pltpu.VMEM((K, BLK), _i32),
            pltpu.VMEM((K, BLK), _i32),
            pltpu.VMEM((K, BLK), _i32),
            pltpu.VMEM((BLK, ncols), _f32),
            pltpu.VMEM((BLK, ncols), _f32),
            pltpu.VMEM((BLK, 16), _f32),
            pltpu.VMEM((BLK, 16), _f32),
            pltpu.VMEM((BLK * 16,), _f32),
            pltpu.SemaphoreType.DMA,
            pltpu.SemaphoreType.DMA,
            pltpu.SemaphoreType.DMA,
            pltpu.SemaphoreType.DMA,
            pltpu.SemaphoreType.DMA,
            pltpu.SemaphoreType.DMA,
            pltpu.SemaphoreType.DMA,
            pltpu.SemaphoreType.DMA,
        ],
    )


def kernel(x, edge_index, W1, att_src1, att_dst1, b1, W2, att_src2, att_dst2,
           b2):
    loop = jnp.arange(N, dtype=_i32)
    src = jnp.concatenate([edge_index[0].astype(_i32), loop,
                           jnp.zeros((EP - NE,), _i32)]).reshape(EP // BLK, BLK)
    dst = jnp.concatenate([edge_index[1].astype(_i32), loop,
                           jnp.full((EP - NE,), N, _i32)]).reshape(EP // BLK, BLK)

    hsel = (jnp.arange(64)[:, None] // 8) == jnp.arange(8)[None, :]
    asrc_m = jnp.where(hsel, att_src1.reshape(64)[:, None], 0.0).astype(_f32)
    adst_m = jnp.where(hsel, att_dst1.reshape(64)[:, None], 0.0).astype(_f32)
    rmap = hsel.T.astype(_f32)

    grid = (10,)
    blkN = N // grid[0]

    table1, adst1 = _tc_call(
        _t1_body,
        (x, W1, asrc_m, adst_m),
        (jax.ShapeDtypeStruct((N, 80), _f32), jax.ShapeDtypeStruct((N, 8), _f32)),
        [pl.BlockSpec((blkN, 128), lambda i: (i, 0)),
         pl.BlockSpec((128, 64), lambda i: (0, 0)),
         pl.BlockSpec((64, 8), lambda i: (0, 0)),
         pl.BlockSpec((64, 8), lambda i: (0, 0))],
        [pl.BlockSpec((blkN, 80), lambda i: (i, 0)),
         pl.BlockSpec((blkN, 8), lambda i: (i, 0))],
        grid,
    )
    adst1_p = jnp.pad(adst1, ((0, NACC - N), (0, 8)))

    sc1 = _make_sc_layer(80, True)
    acc1 = sc1(table1, adst1_p, src, dst)

    table2, adst2 = _tc_call(
        _t2_body,
        (acc1[0, :N], acc1[1, :N], b1.reshape(1, 64), W2, rmap,
         att_src2.reshape(128, 1), att_dst2.reshape(128, 1)),
        (jax.ShapeDtypeStruct((N, 144), _f32), jax.ShapeDtypeStruct((N, 1), _f32)),
        [pl.BlockSpec((blkN, 80), lambda i: (i, 0)),
         pl.BlockSpec((blkN, 80), lambda i: (i, 0)),
         pl.BlockSpec((1, 64), lambda i: (0, 0)),
         pl.BlockSpec((64, 128), lambda i: (0, 0)),
         pl.BlockSpec((8, 64), lambda i: (0, 0)),
         pl.BlockSpec((128, 1), lambda i: (0, 0)),
         pl.BlockSpec((128, 1), lambda i: (0, 0))],
        [pl.BlockSpec((blkN, 144), lambda i: (i, 0)),
         pl.BlockSpec((blkN, 1), lambda i: (i, 0))],
        grid,
    )
    adst2_p = jnp.pad(adst2, ((0, NACC - N), (0, 15)))

    sc2 = _make_sc_layer(144, False)
    acc2 = sc2(table2, adst2_p, src, dst)

    out = _tc_call(
        _t3_body,
        (acc2[0, :N], acc2[1, :N], b2.reshape(1, 128)),
        jax.ShapeDtypeStruct((N, 128), _f32),
        [pl.BlockSpec((blkN, 144), lambda i: (i, 0)),
         pl.BlockSpec((blkN, 144), lambda i: (i, 0)),
         pl.BlockSpec((1, 128), lambda i: (0, 0))],
        pl.BlockSpec((blkN, 128), lambda i: (i, 0)),
        grid,
    )
    return out


# speedup vs baseline: 1.1458x; 1.1458x over previous
import functools

import jax
import jax.numpy as jnp
from jax import lax
from jax.experimental import pallas as pl
from jax.experimental.pallas import tpu as pltpu
from jax.experimental.pallas import tpu_sc as plsc

N = 10000
E = 320000
NE = E + N
NC, NS, L = 2, 16, 16
W = NC * NS
BLK = 96
K = 6
NBLK0 = 132
NBLK1 = 84
EP = (NBLK0 + NBLK1) * NS * BLK
NACC = 10240
TROWS = NACC // NS

_f32 = jnp.float32
_i32 = jnp.int32


def _t1_body(x_ref, w1_ref, asrc_ref, adst_ref, table_ref, adstv_ref):
    h = jnp.dot(x_ref[...], w1_ref[...], preferred_element_type=_f32)
    table_ref[:, 0:64] = h
    table_ref[:, 64:72] = jnp.dot(h, asrc_ref[...], preferred_element_type=_f32)
    table_ref[:, 72:80] = jnp.ones((h.shape[0], 8), _f32)
    adstv_ref[...] = jnp.dot(h, adst_ref[...], preferred_element_type=_f32)


def _t2_body(p0_ref, p1_ref, b1_ref, w2_ref, rmap_ref, as2_ref, ad2_ref,
             table_ref, adstv_ref):
    acc = p0_ref[...] + p1_ref[...]
    recip = 1.0 / acc[:, 72:80]
    out1 = acc[:, 0:64] * jnp.dot(recip, rmap_ref[...],
                                  preferred_element_type=_f32) + b1_ref[...]
    hact = jnp.where(out1 > 0, out1, jnp.exp(out1) - 1.0)
    h2 = jnp.dot(hact, w2_ref[...], preferred_element_type=_f32)
    table_ref[:, 0:128] = h2
    table_ref[:, 128:129] = jnp.dot(h2, as2_ref[...], preferred_element_type=_f32)
    table_ref[:, 129:130] = jnp.ones((h2.shape[0], 1), _f32)
    table_ref[:, 130:144] = jnp.zeros((h2.shape[0], 14), _f32)
    adstv_ref[...] = jnp.dot(h2, ad2_ref[...], preferred_element_type=_f32)


def _t3_body(p0_ref, p1_ref, b2_ref, out_ref):
    acc = p0_ref[...] + p1_ref[...]
    out_ref[...] = acc[:, 0:128] / acc[:, 129:130] + b2_ref[...]


def _tc_call(body, ins, out_shapes, in_blocks, out_blocks, grid):
    return pl.pallas_call(
        body,
        grid=grid,
        in_specs=in_blocks,
        out_specs=out_blocks,
        out_shape=out_shapes,
    )(*ins)


def _sc_edge_kernel(table, adst, src_p, dst_p, out, acc,
                    sidx0, sidx1, didx0, didx1, rows0, rows1, adb0, adb1,
                    alpha1, gsemT0, gsemT1, gsemA0, gsemA1, ssem0, ssem1,
                    isem0, isem1, *, ncols, nvec, layer1):
    c = lax.axis_index("c")
    s = lax.axis_index("s")
    iota = lax.iota(_i32, L)
    sidx = (sidx0, sidx1)
    didx = (didx0, didx1)
    rows = (rows0, rows1)
    adb = (adb0, adb1)
    gsemT = (gsemT0, gsemT1)
    gsemA = (gsemA0, gsemA1)
    ssem = (ssem0, ssem1)
    isem = (isem0, isem1)

    zero = jnp.zeros((L,), _f32)

    def _zr(r, carry):
        for j in range(nvec):
            rows0[r, pl.ds(16 * j, 16)] = zero
        return carry

    lax.fori_loop(0, BLK, _zr, 0)
    tb = s * TROWS
    nz = TROWS // BLK
    for z in range(nz):
        pltpu.sync_copy(rows0, acc.at[pl.ds(tb + z * BLK, BLK)])
    rem = TROWS - nz * BLK
    if rem:
        pltpu.sync_copy(rows0.at[pl.ds(0, rem)],
                        acc.at[pl.ds(tb + nz * BLK, rem)])
    plsc.subcore_barrier()

    sbrow = jnp.where(c == 0, s * NBLK0, NS * NBLK0 + s * NBLK1)
    nsb_half = jnp.where(c == 0, NBLK0 // K // 2, NBLK1 // K // 2)
    if layer1:
        colpats = [2 * j + lax.shift_right_logical(iota, 3) for j in range(4)]
        colpats.append(iota & 7)

    def _sb_load(sb, q, sem_q):
        pltpu.async_copy(src_p.at[pl.ds(sbrow + sb * K, K)], sidx[q], sem_q)
        pltpu.async_copy(dst_p.at[pl.ds(sbrow + sb * K, K)], didx[q], sem_q)

    def _sb_wait(q):
        pltpu.make_async_copy(src_p.at[pl.ds(0, K)], sidx[q], isem[q]).wait()
        pltpu.make_async_copy(dst_p.at[pl.ds(0, K)], didx[q], isem[q]).wait()

    def _gather_issue(q, k, p):
        pltpu.async_copy(table.at[sidx[q].at[k]], rows[p], gsemT[p])
        pltpu.async_copy(adst.at[didx[q].at[k]], adb[p], gsemA[p])

    def _gather_wait(p):
        pltpu.make_async_copy(table.at[sidx[0].at[0]], rows[p], gsemT[p]).wait()
        pltpu.make_async_copy(adst.at[didx[0].at[0]], adb[p], gsemA[p]).wait()

    def _scatter_issue(q, k, p):
        pltpu.async_copy(rows[p], acc.at[didx[q].at[k]], ssem[p], add=True)

    def _scatter_wait(p):
        pltpu.make_async_copy(rows[p], acc.at[didx[0].at[0]], ssem[p]).wait()

    def _compute(p):
        rp, ap = rows[p], adb[p]

        @plsc.parallel_loop(0, BLK, unroll=4)
        def _edge(e):
            a_s = rp[e, pl.ds(64 if layer1 else 128, 16)]
            a_d = ap[e, pl.ds(0, 16)]
            ssum = a_s + a_d
            alpha = jnp.exp(jnp.where(ssum > 0, ssum, 0.2 * ssum))
            alpha1[pl.ds(e * 16, 16)] = alpha
            if layer1:
                for j in range(nvec - 1):
                    av = plsc.load_gather(alpha1, [e * 16 + colpats[j]])
                    rp[e, pl.ds(16 * j, 16)] = rp[e, pl.ds(16 * j, 16)] * av
                av4 = plsc.load_gather(alpha1, [e * 16 + colpats[nvec - 1]])
                rp[e, pl.ds(16 * (nvec - 1), 16)] = av4
            else:
                av = plsc.load_gather(alpha1, [jnp.full((L,), e * 16, _i32)])
                for j in range(nvec - 1):
                    rp[e, pl.ds(16 * j, 16)] = rp[e, pl.ds(16 * j, 16)] * av
                rp[e, pl.ds(16 * (nvec - 1), 16)] = av

    _sb_load(0, 0, isem[0])
    _sb_wait(0)
    _sb_load(1, 1, isem[1])
    _gather_issue(0, 0, 0)

    def _t_step(t, carry):
        for qq in (0, 1):
            sb = 2 * t + qq
            for k in range(K):
                p = k % 2
                _gather_wait(p)
                if k == 0:
                    if qq == 0:
                        @pl.when(t > 0)
                        def _():
                            _scatter_wait(1 - p)
                            _sb_load(sb + 1, 1 - qq, isem[1 - qq])
                    else:
                        _scatter_wait(1 - p)

                        @pl.when(t < nsb_half - 1)
                        def _():
                            _sb_load(sb + 1, 1 - qq, isem[1 - qq])
                else:
                    _scatter_wait(1 - p)
                if k < K - 1:
                    _gather_issue(qq, k + 1, 1 - p)
                else:
                    if qq == 0:
                        _sb_wait(1 - qq)
                        _gather_issue(1 - qq, 0, 1 - p)
                    else:
                        @pl.when(t < nsb_half - 1)
                        def _():
                            _sb_wait(1 - qq)
                            _gather_issue(1 - qq, 0, 1 - p)
                _compute(p)
                _scatter_issue(qq, k, p)
        return carry

    lax.fori_loop(0, nsb_half, _t_step, 0)
    _scatter_wait(1)
    plsc.subcore_barrier()

    pltpu.sync_copy(acc.at[pl.ds(tb, TROWS)], out.at[c, pl.ds(tb, TROWS)])


def _make_sc_layer(ncols, layer1):
    nvec = ncols // 16
    mesh = plsc.VectorSubcoreMesh(core_axis_name="c", subcore_axis_name="s")
    return pl.kernel(
        functools.partial(_sc_edge_kernel, ncols=ncols, nvec=nvec,
                          layer1=layer1),
        out_type=jax.ShapeDtypeStruct((NC, NACC, ncols), _f32),
        mesh=mesh,
        compiler_params=pltpu.CompilerParams(needs_layout_passes=False,
                                             use_tc_tiling_on_sc=False),
        scratch_types=[
            pltpu.VMEM_SHARED((NACC, ncols), _f32),
            pltpu.VMEM((K, BLK), _i32),
            pltpu.VMEM((K, BLK), _i32),
            pltpu.VMEM((K, BLK), _i32),
            pltpu.VMEM((K, BLK), _i32),
            pltpu.VMEM((BLK, ncols), _f32),
            pltpu.VMEM((BLK, ncols), _f32),
            pltpu.VMEM((BLK, 16), _f32),
            pltpu.VMEM((BLK, 16), _f32),
            pltpu.VMEM((BLK * 16,), _f32),
            pltpu.SemaphoreType.DMA,
            pltpu.SemaphoreType.DMA,
            pltpu.SemaphoreType.DMA,
            pltpu.SemaphoreType.DMA,
            pltpu.SemaphoreType.DMA,
            pltpu.SemaphoreType.DMA,
            pltpu.SemaphoreType.DMA,
            pltpu.SemaphoreType.DMA,
        ],
    )


def kernel(x, edge_index, W1, att_src1, att_dst1, b1, W2, att_src2, att_dst2,
           b2):
    loop = jnp.arange(N, dtype=_i32)
    src = jnp.concatenate([edge_index[0].astype(_i32), loop,
                           jnp.zeros((EP - NE,), _i32)]).reshape(EP // BLK, BLK)
    dst = jnp.concatenate([edge_index[1].astype(_i32), loop,
                           jnp.full((EP - NE,), N, _i32)]).reshape(EP // BLK, BLK)

    hsel = (jnp.arange(64)[:, None] // 8) == jnp.arange(8)[None, :]
    asrc_m = jnp.where(hsel, att_src1.reshape(64)[:, None], 0.0).astype(_f32)
    adst_m = jnp.where(hsel, att_dst1.reshape(64)[:, None], 0.0).astype(_f32)
    rmap = hsel.T.astype(_f32)

    grid = (10,)
    blkN = N // grid[0]

    table1, adst1 = _tc_call(
        _t1_body,
        (x, W1, asrc_m, adst_m),
        (jax.ShapeDtypeStruct((N, 80), _f32), jax.ShapeDtypeStruct((N, 8), _f32)),
        [pl.BlockSpec((blkN, 128), lambda i: (i, 0)),
         pl.BlockSpec((128, 64), lambda i: (0, 0)),
         pl.BlockSpec((64, 8), lambda i: (0, 0)),
         pl.BlockSpec((64, 8), lambda i: (0, 0))],
        [pl.BlockSpec((blkN, 80), lambda i: (i, 0)),
         pl.BlockSpec((blkN, 8), lambda i: (i, 0))],
        grid,
    )
    adst1_p = jnp.pad(adst1, ((0, NACC - N), (0, 8)))

    sc1 = _make_sc_layer(80, True)
    acc1 = sc1(table1, adst1_p, src, dst)

    table2, adst2 = _tc_call(
        _t2_body,
        (acc1[0, :N], acc1[1, :N], b1.reshape(1, 64), W2, rmap,
         att_src2.reshape(128, 1), att_dst2.reshape(128, 1)),
        (jax.ShapeDtypeStruct((N, 144), _f32), jax.ShapeDtypeStruct((N, 1), _f32)),
        [pl.BlockSpec((blkN, 80), lambda i: (i, 0)),
         pl.BlockSpec((blkN, 80), lambda i: (i, 0)),
         pl.BlockSpec((1, 64), lambda i: (0, 0)),
         pl.BlockSpec((64, 128), lambda i: (0, 0)),
         pl.BlockSpec((8, 64), lambda i: (0, 0)),
         pl.BlockSpec((128, 1), lambda i: (0, 0)),
         pl.BlockSpec((128, 1), lambda i: (0, 0))],
        [pl.BlockSpec((blkN, 144), lambda i: (i, 0)),
         pl.BlockSpec((blkN, 1), lambda i: (i, 0))],
        grid,
    )
    adst2_p = jnp.pad(adst2, ((0, NACC - N), (0, 15)))

    sc2 = _make_sc_layer(144, False)
    acc2 = sc2(table2, adst2_p, src, dst)

    out = _tc_call(
        _t3_body,
        (acc2[0, :N], acc2[1, :N], b2.reshape(1, 128)),
        jax.ShapeDtypeStruct((N, 128), _f32),
        [pl.BlockSpec((blkN, 144), lambda i: (i, 0)),
         pl.BlockSpec((blkN, 144), lambda i: (i, 0)),
         pl.BlockSpec((1, 128), lambda i: (0, 0))],
        pl.BlockSpec((blkN, 128), lambda i: (i, 0)),
        grid,
    )
    return out


# speedup vs baseline: 1.2308x; 1.0742x over previous
import functools

import jax
import jax.numpy as jnp
from jax import lax
from jax.experimental import pallas as pl
from jax.experimental.pallas import tpu as pltpu
from jax.experimental.pallas import tpu_sc as plsc

N = 10000
E = 320000
NE = E + N
NC, NS, L = 2, 16, 16
W = NC * NS
BLK = 96
K = 6
NBLK0 = 132
NBLK1 = 84
EP = (NBLK0 + NBLK1) * NS * BLK
NACC = 10240
TROWS = NACC // NS

_f32 = jnp.float32
_i32 = jnp.int32


def _t1_body(x_ref, w1_ref, asf_ref, adf_ref, table_ref, adstv_ref):
    h = jnp.dot(x_ref[...], w1_ref[...], preferred_element_type=_f32)
    hs = _hsel()
    table_ref[:, 0:64] = h
    table_ref[:, 64:72] = jnp.dot(h * asf_ref[...], hs,
                                  preferred_element_type=_f32)
    table_ref[:, 72:80] = jnp.ones((h.shape[0], 8), _f32)
    adstv_ref[...] = jnp.dot(h * adf_ref[...], hs, preferred_element_type=_f32)


def _hsel(transpose=False):
    a = lax.broadcasted_iota(_i32, (64, 8), 0) // 8
    b = lax.broadcasted_iota(_i32, (64, 8), 1)
    m = (a == b).astype(_f32)
    return m.T if transpose else m


def _t2_body(p0_ref, p1_ref, b1_ref, w2_ref, as2_ref, ad2_ref,
             table_ref, asrcv_ref, adstv_ref):
    acc = p0_ref[...] + p1_ref[...]
    recip = 1.0 / acc[:, 72:80]
    out1 = acc[:, 0:64] * jnp.dot(recip, _hsel(True),
                                  preferred_element_type=_f32) + b1_ref[...]
    hact = jnp.where(out1 > 0, out1, jnp.exp(out1) - 1.0)
    h2 = jnp.dot(hact, w2_ref[...], preferred_element_type=_f32)
    table_ref[...] = h2
    nb = h2.shape[0]
    asrcv_ref[:, 0:1] = jnp.dot(h2, as2_ref[...], preferred_element_type=_f32)
    asrcv_ref[:, 1:16] = jnp.zeros((nb, 15), _f32)
    adstv_ref[:, 0:1] = jnp.dot(h2, ad2_ref[...], preferred_element_type=_f32)
    adstv_ref[:, 1:16] = jnp.zeros((nb, 15), _f32)


def _t3_body(m0_ref, m1_ref, d0_ref, d1_ref, b2_ref, out_ref):
    msg = m0_ref[...] + m1_ref[...]
    den = d0_ref[:, 0:1] + d1_ref[:, 0:1]
    out_ref[...] = msg / den + b2_ref[...]


def _tc_call(body, ins, out_shapes, in_blocks, out_blocks, grid):
    return pl.pallas_call(
        body,
        grid=grid,
        in_specs=in_blocks,
        out_specs=out_blocks,
        out_shape=out_shapes,
    )(*ins)


def _sc_edge_kernel(table, adst, src_p, dst_p, out, acc,
                    sidx0, sidx1, didx0, didx1, rows0, rows1, adb0, adb1,
                    alpha1, gsemT0, gsemT1, gsemA0, gsemA1, ssem0, ssem1,
                    isem0, isem1, *, ncols, nvec, layer1):
    c = lax.axis_index("c")
    s = lax.axis_index("s")
    iota = lax.iota(_i32, L)
    sidx = (sidx0, sidx1)
    didx = (didx0, didx1)
    rows = (rows0, rows1)
    adb = (adb0, adb1)
    gsemT = (gsemT0, gsemT1)
    gsemA = (gsemA0, gsemA1)
    ssem = (ssem0, ssem1)
    isem = (isem0, isem1)

    zero = jnp.zeros((L,), _f32)

    def _zr(r, carry):
        for j in range(nvec):
            rows0[r, pl.ds(16 * j, 16)] = zero
        return carry

    lax.fori_loop(0, BLK, _zr, 0)
    tb = s * TROWS
    nz = TROWS // BLK
    for z in range(nz):
        pltpu.sync_copy(rows0, acc.at[pl.ds(tb + z * BLK, BLK)])
    rem = TROWS - nz * BLK
    if rem:
        pltpu.sync_copy(rows0.at[pl.ds(0, rem)],
                        acc.at[pl.ds(tb + nz * BLK, rem)])
    plsc.subcore_barrier()

    sbrow = jnp.where(c == 0, s * NBLK0, NS * NBLK0 + s * NBLK1)
    nsb_half = jnp.where(c == 0, NBLK0 // K // 2, NBLK1 // K // 2)
    if layer1:
        colpats = [2 * j + lax.shift_right_logical(iota, 3) for j in range(4)]
        colpats.append(iota & 7)

    def _sb_load(sb, q, sem_q):
        pltpu.async_copy(src_p.at[pl.ds(sbrow + sb * K, K)], sidx[q], sem_q)
        pltpu.async_copy(dst_p.at[pl.ds(sbrow + sb * K, K)], didx[q], sem_q)

    def _sb_wait(q):
        pltpu.make_async_copy(src_p.at[pl.ds(0, K)], sidx[q], isem[q]).wait()
        pltpu.make_async_copy(dst_p.at[pl.ds(0, K)], didx[q], isem[q]).wait()

    def _gather_issue(q, k, p):
        pltpu.async_copy(table.at[sidx[q].at[k]], rows[p], gsemT[p])
        pltpu.async_copy(adst.at[didx[q].at[k]], adb[p], gsemA[p])

    def _gather_wait(p):
        pltpu.make_async_copy(table.at[sidx[0].at[0]], rows[p], gsemT[p]).wait()
        pltpu.make_async_copy(adst.at[didx[0].at[0]], adb[p], gsemA[p]).wait()

    def _scatter_issue(q, k, p):
        pltpu.async_copy(rows[p], acc.at[didx[q].at[k]], ssem[p], add=True)

    def _scatter_wait(p):
        pltpu.make_async_copy(rows[p], acc.at[didx[0].at[0]], ssem[p]).wait()

    def _compute(p):
        rp, ap = rows[p], adb[p]

        @plsc.parallel_loop(0, BLK, unroll=4)
        def _edge(e):
            a_s = rp[e, pl.ds(64 if layer1 else 128, 16)]
            a_d = ap[e, pl.ds(0, 16)]
            ssum = a_s + a_d
            alpha = jnp.exp(jnp.where(ssum > 0, ssum, 0.2 * ssum))
            alpha1[pl.ds(e * 16, 16)] = alpha
            if layer1:
                for j in range(nvec - 1):
                    av = plsc.load_gather(alpha1, [e * 16 + colpats[j]])
                    rp[e, pl.ds(16 * j, 16)] = rp[e, pl.ds(16 * j, 16)] * av
                av4 = plsc.load_gather(alpha1, [e * 16 + colpats[nvec - 1]])
                rp[e, pl.ds(16 * (nvec - 1), 16)] = av4
            else:
                av = plsc.load_gather(alpha1, [jnp.full((L,), e * 16, _i32)])
                for j in range(nvec - 1):
                    rp[e, pl.ds(16 * j, 16)] = rp[e, pl.ds(16 * j, 16)] * av
                rp[e, pl.ds(16 * (nvec - 1), 16)] = av

    _sb_load(0, 0, isem[0])
    _sb_wait(0)
    _sb_load(1, 1, isem[1])
    _gather_issue(0, 0, 0)

    def _t_step(t, carry):
        for qq in (0, 1):
            sb = 2 * t + qq
            for k in range(K):
                p = k % 2
                _gather_wait(p)
                if k == 0:
                    if qq == 0:
                        @pl.when(t > 0)
                        def _():
                            _scatter_wait(1 - p)
                            _sb_load(sb + 1, 1 - qq, isem[1 - qq])
                    else:
                        _scatter_wait(1 - p)

                        @pl.when(t < nsb_half - 1)
                        def _():
                            _sb_load(sb + 1, 1 - qq, isem[1 - qq])
                else:
                    _scatter_wait(1 - p)
                if k < K - 1:
                    _gather_issue(qq, k + 1, 1 - p)
                else:
                    if qq == 0:
                        _sb_wait(1 - qq)
                        _gather_issue(1 - qq, 0, 1 - p)
                    else:
                        @pl.when(t < nsb_half - 1)
                        def _():
                            _sb_wait(1 - qq)
                            _gather_issue(1 - qq, 0, 1 - p)
                _compute(p)
                _scatter_issue(qq, k, p)
        return carry

    lax.fori_loop(0, nsb_half, _t_step, 0)
    _scatter_wait(1)
    plsc.subcore_barrier()

    pltpu.sync_copy(acc.at[pl.ds(tb, TROWS)], out.at[c, pl.ds(tb, TROWS)])


def _sc_edge_kernel2(table, asrc, adst, src_p, dst_p, outm, outd, accm, accd,
                     sidx0, sidx1, didx0, didx1, rows0, rows1, asb0, asb1,
                     adb0, adb1, dbuf0, dbuf1, alpha1,
                     gsemT0, gsemT1, gsemS0, gsemS1, gsemA0, gsemA1,
                     ssem0, ssem1, dsem0, dsem1, isem0, isem1):
    c = lax.axis_index("c")
    s = lax.axis_index("s")
    iota = lax.iota(_i32, L)
    sidx = (sidx0, sidx1)
    didx = (didx0, didx1)
    rows = (rows0, rows1)
    asb = (asb0, asb1)
    adb = (adb0, adb1)
    dbuf = (dbuf0, dbuf1)
    gsemT = (gsemT0, gsemT1)
    gsemS = (gsemS0, gsemS1)
    gsemA = (gsemA0, gsemA1)
    ssem = (ssem0, ssem1)
    dsem = (dsem0, dsem1)
    isem = (isem0, isem1)

    zero = jnp.zeros((L,), _f32)

    def _zr(r, carry):
        for j in range(8):
            rows0[r, pl.ds(16 * j, 16)] = zero
        dbuf0[r, pl.ds(0, 16)] = zero
        return carry

    lax.fori_loop(0, BLK, _zr, 0)
    tb = s * TROWS
    nz = TROWS // BLK
    for z in range(nz):
        pltpu.sync_copy(rows0, accm.at[pl.ds(tb + z * BLK, BLK)])
        pltpu.sync_copy(dbuf0, accd.at[pl.ds(tb + z * BLK, BLK)])
    rem = TROWS - nz * BLK
    if rem:
        pltpu.sync_copy(rows0.at[pl.ds(0, rem)],
                        accm.at[pl.ds(tb + nz * BLK, rem)])
        pltpu.sync_copy(dbuf0.at[pl.ds(0, rem)],
                        accd.at[pl.ds(tb + nz * BLK, rem)])
    plsc.subcore_barrier()

    sbrow = jnp.where(c == 0, s * NBLK0, NS * NBLK0 + s * NBLK1)
    nsb_half = jnp.where(c == 0, NBLK0 // K // 2, NBLK1 // K // 2)

    def _sb_load(sb, q, sem_q):
        pltpu.async_copy(src_p.at[pl.ds(sbrow + sb * K, K)], sidx[q], sem_q)
        pltpu.async_copy(dst_p.at[pl.ds(sbrow + sb * K, K)], didx[q], sem_q)

    def _sb_wait(q):
        pltpu.make_async_copy(src_p.at[pl.ds(0, K)], sidx[q], isem[q]).wait()
        pltpu.make_async_copy(dst_p.at[pl.ds(0, K)], didx[q], isem[q]).wait()

    def _gather_issue(q, k, p):
        pltpu.async_copy(table.at[sidx[q].at[k]], rows[p], gsemT[p])
        pltpu.async_copy(asrc.at[sidx[q].at[k]], asb[p], gsemS[p])
        pltpu.async_copy(adst.at[didx[q].at[k]], adb[p], gsemA[p])

    def _gather_wait(p):
        pltpu.make_async_copy(table.at[sidx[0].at[0]], rows[p], gsemT[p]).wait()
        pltpu.make_async_copy(asrc.at[sidx[0].at[0]], asb[p], gsemS[p]).wait()
        pltpu.make_async_copy(adst.at[didx[0].at[0]], adb[p], gsemA[p]).wait()

    def _scatter_issue(q, k, p):
        pltpu.async_copy(rows[p], accm.at[didx[q].at[k]], ssem[p], add=True)
        pltpu.async_copy(dbuf[p], accd.at[didx[q].at[k]], dsem[p], add=True)

    def _scatter_wait(p):
        pltpu.make_async_copy(rows[p], accm.at[didx[0].at[0]], ssem[p]).wait()
        pltpu.make_async_copy(dbuf[p], accd.at[didx[0].at[0]], dsem[p]).wait()

    def _compute(p):
        rp, sp, ap, dp = rows[p], asb[p], adb[p], dbuf[p]

        @plsc.parallel_loop(0, BLK, unroll=4)
        def _edge(e):
            ssum = sp[e, pl.ds(0, 16)] + ap[e, pl.ds(0, 16)]
            alpha = jnp.exp(jnp.where(ssum > 0, ssum, 0.2 * ssum))
            alpha1[pl.ds(e * 16, 16)] = alpha
            av = plsc.load_gather(alpha1, [jnp.full((L,), e * 16, _i32)])
            for j in range(8):
                rp[e, pl.ds(16 * j, 16)] = rp[e, pl.ds(16 * j, 16)] * av
            dp[e, pl.ds(0, 16)] = av

    _sb_load(0, 0, isem[0])
    _sb_wait(0)
    _sb_load(1, 1, isem[1])
    _gather_issue(0, 0, 0)

    def _t_step(t, carry):
        for qq in (0, 1):
            sb = 2 * t + qq
            for k in range(K):
                p = k % 2
                _gather_wait(p)
                if k == 0:
                    if qq == 0:
                        @pl.when(t > 0)
                        def _():
                            _scatter_wait(1 - p)
                            _sb_load(sb + 1, 1 - qq, isem[1 - qq])
                    else:
                        _scatter_wait(1 - p)

                        @pl.when(t < nsb_half - 1)
                        def _():
                            _sb_load(sb + 1, 1 - qq, isem[1 - qq])
                else:
                    _scatter_wait(1 - p)
                if k < K - 1:
                    _gather_issue(qq, k + 1, 1 - p)
                else:
                    if qq == 0:
                        _sb_wait(1 - qq)
                        _gather_issue(1 - qq, 0, 1 - p)
                    else:
                        @pl.when(t < nsb_half - 1)
                        def _():
                            _sb_wait(1 - qq)
                            _gather_issue(1 - qq, 0, 1 - p)
                _compute(p)
                _scatter_issue(qq, k, p)
        return carry

    lax.fori_loop(0, nsb_half, _t_step, 0)
    _scatter_wait(1)
    plsc.subcore_barrier()

    pltpu.sync_copy(accm.at[pl.ds(tb, TROWS)], outm.at[c, pl.ds(tb, TROWS)])
    pltpu.sync_copy(accd.at[pl.ds(tb, TROWS)], outd.at[c, pl.ds(tb, TROWS)])


def _make_sc_layer2():
    mesh = plsc.VectorSubcoreMesh(core_axis_name="c", subcore_axis_name="s")
    return pl.kernel(
        _sc_edge_kernel2,
        out_type=(jax.ShapeDtypeStruct((NC, NACC, 128), _f32),
                  jax.ShapeDtypeStruct((NC, NACC, 16), _f32)),
        mesh=mesh,
        compiler_params=pltpu.CompilerParams(needs_layout_passes=False,
                                             use_tc_tiling_on_sc=False),
        scratch_types=(
            [pltpu.VMEM_SHARED((NACC, 128), _f32),
             pltpu.VMEM_SHARED((NACC, 16), _f32)]
            + [pltpu.VMEM((K, BLK), _i32)] * 4
            + [pltpu.VMEM((BLK, 128), _f32)] * 2
            + [pltpu.VMEM((BLK, 16), _f32)] * 4
            + [pltpu.VMEM((BLK, 16), _f32)] * 2
            + [pltpu.VMEM((BLK * 16,), _f32)]
            + [pltpu.SemaphoreType.DMA] * 12
        ),
    )


def _make_sc_layer(ncols, layer1):
    nvec = ncols // 16
    mesh = plsc.VectorSubcoreMesh(core_axis_name="c", subcore_axis_name="s")
    return pl.kernel(
        functools.partial(_sc_edge_kernel, ncols=ncols, nvec=nvec,
                          layer1=layer1),
        out_type=jax.ShapeDtypeStruct((NC, NACC, ncols), _f32),
        mesh=mesh,
        compiler_params=pltpu.CompilerParams(needs_layout_passes=False,
                                             use_tc_tiling_on_sc=False),
        scratch_types=[
            pltpu.VMEM_SHARED((NACC, ncols), _f32),
            pltpu.VMEM((K, BLK), _i32),
            pltpu.VMEM((K, BLK), _i32),
            pltpu.VMEM((K, BLK), _i32),
            pltpu.VMEM((K, BLK), _i32),
            pltpu.VMEM((BLK, ncols), _f32),
            pltpu.VMEM((BLK, ncols), _f32),
            pltpu.VMEM((BLK, 16), _f32),
            pltpu.VMEM((BLK, 16), _f32),
            pltpu.VMEM((BLK * 16,), _f32),
            pltpu.SemaphoreType.DMA,
            pltpu.SemaphoreType.DMA,
            pltpu.SemaphoreType.DMA,
            pltpu.SemaphoreType.DMA,
            pltpu.SemaphoreType.DMA,
            pltpu.SemaphoreType.DMA,
            pltpu.SemaphoreType.DMA,
            pltpu.SemaphoreType.DMA,
        ],
    )


def kernel(x, edge_index, W1, att_src1, att_dst1, b1, W2, att_src2, att_dst2,
           b2):
    loop = jnp.arange(N, dtype=_i32)
    src = jnp.concatenate([edge_index[0].astype(_i32), loop,
                           jnp.zeros((EP - NE,), _i32)]).reshape(EP // BLK, BLK)
    dst = jnp.concatenate([edge_index[1].astype(_i32), loop,
                           jnp.full((EP - NE,), N, _i32)]).reshape(EP // BLK, BLK)

    grid = (10,)
    blkN = N // grid[0]

    table1, adst1 = _tc_call(
        _t1_body,
        (x, W1, att_src1.reshape(1, 64), att_dst1.reshape(1, 64)),
        (jax.ShapeDtypeStruct((N, 80), _f32), jax.ShapeDtypeStruct((N, 8), _f32)),
        [pl.BlockSpec((blkN, 128), lambda i: (i, 0)),
         pl.BlockSpec((128, 64), lambda i: (0, 0)),
         pl.BlockSpec((1, 64), lambda i: (0, 0)),
         pl.BlockSpec((1, 64), lambda i: (0, 0))],
        [pl.BlockSpec((blkN, 80), lambda i: (i, 0)),
         pl.BlockSpec((blkN, 8), lambda i: (i, 0))],
        grid,
    )
    adst1_p = jnp.pad(adst1, ((0, NACC - N), (0, 8)))

    sc1 = _make_sc_layer(80, True)
    acc1 = sc1(table1, adst1_p, src, dst)

    table2, asrc2, adst2 = _tc_call(
        _t2_body,
        (acc1[0, :N], acc1[1, :N], b1.reshape(1, 64), W2,
         att_src2.reshape(128, 1), att_dst2.reshape(128, 1)),
        (jax.ShapeDtypeStruct((N, 128), _f32),
         jax.ShapeDtypeStruct((N, 16), _f32),
         jax.ShapeDtypeStruct((N, 16), _f32)),
        [pl.BlockSpec((blkN, 80), lambda i: (i, 0)),
         pl.BlockSpec((blkN, 80), lambda i: (i, 0)),
         pl.BlockSpec((1, 64), lambda i: (0, 0)),
         pl.BlockSpec((64, 128), lambda i: (0, 0)),
         pl.BlockSpec((128, 1), lambda i: (0, 0)),
         pl.BlockSpec((128, 1), lambda i: (0, 0))],
        [pl.BlockSpec((blkN, 128), lambda i: (i, 0)),
         pl.BlockSpec((blkN, 16), lambda i: (i, 0)),
         pl.BlockSpec((blkN, 16), lambda i: (i, 0))],
        grid,
    )
    asrc2_p = jnp.pad(asrc2, ((0, NACC - N), (0, 0)))
    adst2_p = jnp.pad(adst2, ((0, NACC - N), (0, 0)))

    sc2 = _make_sc_layer2()
    accm, accd = sc2(table2, asrc2_p, adst2_p, src, dst)

    out = _tc_call(
        _t3_body,
        (accm[0, :N], accm[1, :N], accd[0, :N], accd[1, :N],
         b2.reshape(1, 128)),
        jax.ShapeDtypeStruct((N, 128), _f32),
        [pl.BlockSpec((blkN, 128), lambda i: (i, 0)),
         pl.BlockSpec((blkN, 128), lambda i: (i, 0)),
         pl.BlockSpec((blkN, 16), lambda i: (i, 0)),
         pl.BlockSpec((blkN, 16), lambda i: (i, 0)),
         pl.BlockSpec((1, 128), lambda i: (0, 0))],
        pl.BlockSpec((blkN, 128), lambda i: (i, 0)),
        grid,
    )
    return out


# speedup vs baseline: 1.2615x; 1.0249x over previous
import functools

import jax
import jax.numpy as jnp
from jax import lax
from jax.experimental import pallas as pl
from jax.experimental.pallas import tpu as pltpu
from jax.experimental.pallas import tpu_sc as plsc

N = 10000
E = 320000
NE = E + N
NC, NS, L = 2, 16, 16
W = NC * NS
BLK = 96
K = 6
NBLK0 = 132
NBLK1 = 84
EP = (NBLK0 + NBLK1) * NS * BLK
NACC = 10240
TROWS = NACC // NS

_f32 = jnp.float32
_i32 = jnp.int32


def _t1_body(x_ref, w1_ref, asf_ref, adf_ref, table_ref, adstv_ref):
    h = jnp.dot(x_ref[...], w1_ref[...], preferred_element_type=_f32)
    hs = _hsel()
    nb = h.shape[0]
    table_ref[:, 0:64] = h
    table_ref[:, 64:72] = jnp.dot(h * asf_ref[...], hs,
                                  preferred_element_type=_f32)
    table_ref[:, 72:80] = jnp.ones((nb, 8), _f32)
    adstv_ref[:, 0:8] = jnp.dot(h * adf_ref[...], hs,
                                preferred_element_type=_f32)
    adstv_ref[:, 8:16] = jnp.zeros((nb, 8), _f32)


def _hsel(transpose=False):
    a = lax.broadcasted_iota(_i32, (64, 8), 0) // 8
    b = lax.broadcasted_iota(_i32, (64, 8), 1)
    m = (a == b).astype(_f32)
    return m.T if transpose else m


def _t2_body(p0_ref, p1_ref, b1_ref, w2_ref, as2_ref, ad2_ref,
             table_ref, asrcv_ref, adstv_ref):
    acc = p0_ref[0] + p1_ref[0]
    recip = 1.0 / acc[:, 72:80]
    out1 = acc[:, 0:64] * jnp.dot(recip, _hsel(True),
                                  preferred_element_type=_f32) + b1_ref[...]
    hact = jnp.where(out1 > 0, out1, jnp.exp(out1) - 1.0)
    h2 = jnp.dot(hact, w2_ref[...], preferred_element_type=_f32)
    table_ref[...] = h2
    nb = h2.shape[0]
    asrcv_ref[:, 0:1] = jnp.dot(h2, as2_ref[...], preferred_element_type=_f32)
    asrcv_ref[:, 1:16] = jnp.zeros((nb, 15), _f32)
    adstv_ref[:, 0:1] = jnp.dot(h2, ad2_ref[...], preferred_element_type=_f32)
    adstv_ref[:, 1:16] = jnp.zeros((nb, 15), _f32)


def _t3_body(m0_ref, m1_ref, d0_ref, d1_ref, b2_ref, out_ref):
    msg = m0_ref[0] + m1_ref[0]
    den = d0_ref[0, :, 0:1] + d1_ref[0, :, 0:1]
    out_ref[...] = msg / den + b2_ref[...]


def _tc_call(body, ins, out_shapes, in_blocks, out_blocks, grid):
    return pl.pallas_call(
        body,
        grid=grid,
        in_specs=in_blocks,
        out_specs=out_blocks,
        out_shape=out_shapes,
    )(*ins)


def _sc_edge_kernel(table, adst, src_p, dst_p, out, acc,
                    sidx0, sidx1, didx0, didx1, rows0, rows1, adb0, adb1,
                    alpha1, gsemT0, gsemT1, gsemA0, gsemA1, ssem0, ssem1,
                    isem0, isem1, *, ncols, nvec, layer1):
    c = lax.axis_index("c")
    s = lax.axis_index("s")
    iota = lax.iota(_i32, L)
    sidx = (sidx0, sidx1)
    didx = (didx0, didx1)
    rows = (rows0, rows1)
    adb = (adb0, adb1)
    gsemT = (gsemT0, gsemT1)
    gsemA = (gsemA0, gsemA1)
    ssem = (ssem0, ssem1)
    isem = (isem0, isem1)

    zero = jnp.zeros((L,), _f32)

    def _zr(r, carry):
        for j in range(nvec):
            rows0[r, pl.ds(16 * j, 16)] = zero
        return carry

    lax.fori_loop(0, BLK, _zr, 0)
    tb = s * TROWS
    nz = TROWS // BLK
    for z in range(nz):
        pltpu.sync_copy(rows0, acc.at[pl.ds(tb + z * BLK, BLK)])
    rem = TROWS - nz * BLK
    if rem:
        pltpu.sync_copy(rows0.at[pl.ds(0, rem)],
                        acc.at[pl.ds(tb + nz * BLK, rem)])
    plsc.subcore_barrier()

    sbrow = jnp.where(c == 0, s * NBLK0, NS * NBLK0 + s * NBLK1)
    nsb_half = jnp.where(c == 0, NBLK0 // K // 2, NBLK1 // K // 2)
    if layer1:
        colpats = [2 * j + lax.shift_right_logical(iota, 3) for j in range(4)]
        colpats.append(iota & 7)

    def _sb_load(sb, q, sem_q):
        pltpu.async_copy(src_p.at[pl.ds(sbrow + sb * K, K)], sidx[q], sem_q)
        pltpu.async_copy(dst_p.at[pl.ds(sbrow + sb * K, K)], didx[q], sem_q)

    def _sb_wait(q):
        pltpu.make_async_copy(src_p.at[pl.ds(0, K)], sidx[q], isem[q]).wait()
        pltpu.make_async_copy(dst_p.at[pl.ds(0, K)], didx[q], isem[q]).wait()

    def _gather_issue(q, k, p):
        pltpu.async_copy(table.at[sidx[q].at[k]], rows[p], gsemT[p])
        pltpu.async_copy(adst.at[didx[q].at[k]], adb[p], gsemA[p])

    def _gather_wait(p):
        pltpu.make_async_copy(table.at[sidx[0].at[0]], rows[p], gsemT[p]).wait()
        pltpu.make_async_copy(adst.at[didx[0].at[0]], adb[p], gsemA[p]).wait()

    def _scatter_issue(q, k, p):
        pltpu.async_copy(rows[p], acc.at[didx[q].at[k]], ssem[p], add=True)

    def _scatter_wait(p):
        pltpu.make_async_copy(rows[p], acc.at[didx[0].at[0]], ssem[p]).wait()

    def _compute(p):
        rp, ap = rows[p], adb[p]

        @plsc.parallel_loop(0, BLK, unroll=4)
        def _edge(e):
            a_s = rp[e, pl.ds(64 if layer1 else 128, 16)]
            a_d = ap[e, pl.ds(0, 16)]
            ssum = a_s + a_d
            alpha = jnp.exp(jnp.where(ssum > 0, ssum, 0.2 * ssum))
            alpha1[pl.ds(e * 16, 16)] = alpha
            if layer1:
                for j in range(nvec - 1):
                    av = plsc.load_gather(alpha1, [e * 16 + colpats[j]])
                    rp[e, pl.ds(16 * j, 16)] = rp[e, pl.ds(16 * j, 16)] * av
                av4 = plsc.load_gather(alpha1, [e * 16 + colpats[nvec - 1]])
                rp[e, pl.ds(16 * (nvec - 1), 16)] = av4
            else:
                av = plsc.load_gather(alpha1, [jnp.full((L,), e * 16, _i32)])
                for j in range(nvec - 1):
                    rp[e, pl.ds(16 * j, 16)] = rp[e, pl.ds(16 * j, 16)] * av
                rp[e, pl.ds(16 * (nvec - 1), 16)] = av

    _sb_load(0, 0, isem[0])
    _sb_wait(0)
    _sb_load(1, 1, isem[1])
    _gather_issue(0, 0, 0)

    def _t_step(t, carry):
        for qq in (0, 1):
            sb = 2 * t + qq
            for k in range(K):
                p = k % 2
                _gather_wait(p)
                if k == 0:
                    if qq == 0:
                        @pl.when(t > 0)
                        def _():
                            _scatter_wait(1 - p)
                            _sb_load(sb + 1, 1 - qq, isem[1 - qq])
                    else:
                        _scatter_wait(1 - p)

                        @pl.when(t < nsb_half - 1)
                        def _():
                            _sb_load(sb + 1, 1 - qq, isem[1 - qq])
                else:
                    _scatter_wait(1 - p)
                if k < K - 1:
                    _gather_issue(qq, k + 1, 1 - p)
                else:
                    if qq == 0:
                        _sb_wait(1 - qq)
                        _gather_issue(1 - qq, 0, 1 - p)
                    else:
                        @pl.when(t < nsb_half - 1)
                        def _():
                            _sb_wait(1 - qq)
                            _gather_issue(1 - qq, 0, 1 - p)
                _compute(p)
                _scatter_issue(qq, k, p)
        return carry

    lax.fori_loop(0, nsb_half, _t_step, 0)
    _scatter_wait(1)
    plsc.subcore_barrier()

    pltpu.sync_copy(acc.at[pl.ds(tb, TROWS)], out.at[c, pl.ds(tb, TROWS)])


def _sc_edge_kernel2(table, asrc, adst, src_p, dst_p, outm, outd, accm, accd,
                     sidx0, sidx1, didx0, didx1, rows0, rows1, asb0, asb1,
                     adb0, adb1, dbuf0, dbuf1, alpha1,
                     gsemT0, gsemT1, gsemS0, gsemS1, gsemA0, gsemA1,
                     ssem0, ssem1, dsem0, dsem1, isem0, isem1):
    c = lax.axis_index("c")
    s = lax.axis_index("s")
    iota = lax.iota(_i32, L)
    sidx = (sidx0, sidx1)
    didx = (didx0, didx1)
    rows = (rows0, rows1)
    asb = (asb0, asb1)
    adb = (adb0, adb1)
    dbuf = (dbuf0, dbuf1)
    gsemT = (gsemT0, gsemT1)
    gsemS = (gsemS0, gsemS1)
    gsemA = (gsemA0, gsemA1)
    ssem = (ssem0, ssem1)
    dsem = (dsem0, dsem1)
    isem = (isem0, isem1)

    zero = jnp.zeros((L,), _f32)

    def _zr(r, carry):
        for j in range(8):
            rows0[r, pl.ds(16 * j, 16)] = zero
        dbuf0[r, pl.ds(0, 16)] = zero
        return carry

    lax.fori_loop(0, BLK, _zr, 0)
    tb = s * TROWS
    nz = TROWS // BLK
    for z in range(nz):
        pltpu.sync_copy(rows0, accm.at[pl.ds(tb + z * BLK, BLK)])
        pltpu.sync_copy(dbuf0, accd.at[pl.ds(tb + z * BLK, BLK)])
    rem = TROWS - nz * BLK
    if rem:
        pltpu.sync_copy(rows0.at[pl.ds(0, rem)],
                        accm.at[pl.ds(tb + nz * BLK, rem)])
        pltpu.sync_copy(dbuf0.at[pl.ds(0, rem)],
                        accd.at[pl.ds(tb + nz * BLK, rem)])
    plsc.subcore_barrier()

    sbrow = jnp.where(c == 0, s * NBLK0, NS * NBLK0 + s * NBLK1)
    nsb_half = jnp.where(c == 0, NBLK0 // K // 2, NBLK1 // K // 2)

    def _sb_load(sb, q, sem_q):
        pltpu.async_copy(src_p.at[pl.ds(sbrow + sb * K, K)], sidx[q], sem_q)
        pltpu.async_copy(dst_p.at[pl.ds(sbrow + sb * K, K)], didx[q], sem_q)

    def _sb_wait(q):
        pltpu.make_async_copy(src_p.at[pl.ds(0, K)], sidx[q], isem[q]).wait()
        pltpu.make_async_copy(dst_p.at[pl.ds(0, K)], didx[q], isem[q]).wait()

    def _gather_issue(q, k, p):
        pltpu.async_copy(table.at[sidx[q].at[k]], rows[p], gsemT[p])
        pltpu.async_copy(asrc.at[sidx[q].at[k]], asb[p], gsemS[p])
        pltpu.async_copy(adst.at[didx[q].at[k]], adb[p], gsemA[p])

    def _gather_wait(p):
        pltpu.make_async_copy(table.at[sidx[0].at[0]], rows[p], gsemT[p]).wait()
        pltpu.make_async_copy(asrc.at[sidx[0].at[0]], asb[p], gsemS[p]).wait()
        pltpu.make_async_copy(adst.at[didx[0].at[0]], adb[p], gsemA[p]).wait()

    def _scatter_issue(q, k, p):
        pltpu.async_copy(rows[p], accm.at[didx[q].at[k]], ssem[p], add=True)
        pltpu.async_copy(dbuf[p], accd.at[didx[q].at[k]], dsem[p], add=True)

    def _scatter_wait(p):
        pltpu.make_async_copy(rows[p], accm.at[didx[0].at[0]], ssem[p]).wait()
        pltpu.make_async_copy(dbuf[p], accd.at[didx[0].at[0]], dsem[p]).wait()

    def _compute(p):
        rp, sp, ap, dp = rows[p], asb[p], adb[p], dbuf[p]

        @plsc.parallel_loop(0, BLK, unroll=4)
        def _edge(e):
            ssum = sp[e, pl.ds(0, 16)] + ap[e, pl.ds(0, 16)]
            alpha = jnp.exp(jnp.where(ssum > 0, ssum, 0.2 * ssum))
            alpha1[pl.ds(e * 16, 16)] = alpha
            av = plsc.load_gather(alpha1, [jnp.full((L,), e * 16, _i32)])
            for j in range(8):
                rp[e, pl.ds(16 * j, 16)] = rp[e, pl.ds(16 * j, 16)] * av
            dp[e, pl.ds(0, 16)] = av

    _sb_load(0, 0, isem[0])
    _sb_wait(0)
    _sb_load(1, 1, isem[1])
    _gather_issue(0, 0, 0)

    def _t_step(t, carry):
        for qq in (0, 1):
            sb = 2 * t + qq
            for k in range(K):
                p = k % 2
                _gather_wait(p)
                if k == 0:
                    if qq == 0:
                        @pl.when(t > 0)
                        def _():
                            _scatter_wait(1 - p)
                            _sb_load(sb + 1, 1 - qq, isem[1 - qq])
                    else:
                        _scatter_wait(1 - p)

                        @pl.when(t < nsb_half - 1)
                        def _():
                            _sb_load(sb + 1, 1 - qq, isem[1 - qq])
                else:
                    _scatter_wait(1 - p)
                if k < K - 1:
                    _gather_issue(qq, k + 1, 1 - p)
                else:
                    if qq == 0:
                        _sb_wait(1 - qq)
                        _gather_issue(1 - qq, 0, 1 - p)
                    else:
                        @pl.when(t < nsb_half - 1)
                        def _():
                            _sb_wait(1 - qq)
                            _gather_issue(1 - qq, 0, 1 - p)
                _compute(p)
                _scatter_issue(qq, k, p)
        return carry

    lax.fori_loop(0, nsb_half, _t_step, 0)
    _scatter_wait(1)
    plsc.subcore_barrier()

    pltpu.sync_copy(accm.at[pl.ds(tb, TROWS)], outm.at[c, pl.ds(tb, TROWS)])
    pltpu.sync_copy(accd.at[pl.ds(tb, TROWS)], outd.at[c, pl.ds(tb, TROWS)])


def _make_sc_layer2():
    mesh = plsc.VectorSubcoreMesh(core_axis_name="c", subcore_axis_name="s")
    return pl.kernel(
        _sc_edge_kernel2,
        out_type=(jax.ShapeDtypeStruct((NC, NACC, 128), _f32),
                  jax.ShapeDtypeStruct((NC, NACC, 16), _f32)),
        mesh=mesh,
        compiler_params=pltpu.CompilerParams(needs_layout_passes=False,
                                             use_tc_tiling_on_sc=False),
        scratch_types=(
            [pltpu.VMEM_SHARED((NACC, 128), _f32),
             pltpu.VMEM_SHARED((NACC, 16), _f32)]
            + [pltpu.VMEM((K, BLK), _i32)] * 4
            + [pltpu.VMEM((BLK, 128), _f32)] * 2
            + [pltpu.VMEM((BLK, 16), _f32)] * 4
            + [pltpu.VMEM((BLK, 16), _f32)] * 2
            + [pltpu.VMEM((BLK * 16,), _f32)]
            + [pltpu.SemaphoreType.DMA] * 12
        ),
    )


def _make_sc_layer(ncols, layer1):
    nvec = ncols // 16
    mesh = plsc.VectorSubcoreMesh(core_axis_name="c", subcore_axis_name="s")
    return pl.kernel(
        functools.partial(_sc_edge_kernel, ncols=ncols, nvec=nvec,
                          layer1=layer1),
        out_type=jax.ShapeDtypeStruct((NC, NACC, ncols), _f32),
        mesh=mesh,
        compiler_params=pltpu.CompilerParams(needs_layout_passes=False,
                                             use_tc_tiling_on_sc=False),
        scratch_types=[
            pltpu.VMEM_SHARED((NACC, ncols), _f32),
            pltpu.VMEM((K, BLK), _i32),
            pltpu.VMEM((K, BLK), _i32),
            pltpu.VMEM((K, BLK), _i32),
            pltpu.VMEM((K, BLK), _i32),
            pltpu.VMEM((BLK, ncols), _f32),
            pltpu.VMEM((BLK, ncols), _f32),
            pltpu.VMEM((BLK, 16), _f32),
            pltpu.VMEM((BLK, 16), _f32),
            pltpu.VMEM((BLK * 16,), _f32),
            pltpu.SemaphoreType.DMA,
            pltpu.SemaphoreType.DMA,
            pltpu.SemaphoreType.DMA,
            pltpu.SemaphoreType.DMA,
            pltpu.SemaphoreType.DMA,
            pltpu.SemaphoreType.DMA,
            pltpu.SemaphoreType.DMA,
            pltpu.SemaphoreType.DMA,
        ],
    )


def kernel(x, edge_index, W1, att_src1, att_dst1, b1, W2, att_src2, att_dst2,
           b2):
    idx = jnp.arange(EP, dtype=_i32)
    s_pad = jnp.pad(edge_index[0].astype(_i32), (0, EP - E))
    d_pad = jnp.pad(edge_index[1].astype(_i32), (0, EP - E))
    src = jnp.where(idx < E, s_pad,
                    jnp.where(idx < NE, idx - E, 0)).reshape(EP // BLK, BLK)
    dst = jnp.where(idx < E, d_pad,
                    jnp.where(idx < NE, idx - E, N)).reshape(EP // BLK, BLK)

    grid = (10,)
    blkN = N // grid[0]

    table1, adst1 = _tc_call(
        _t1_body,
        (x, W1, att_src1.reshape(1, 64), att_dst1.reshape(1, 64)),
        (jax.ShapeDtypeStruct((N, 80), _f32),
         jax.ShapeDtypeStruct((NACC, 16), _f32)),
        [pl.BlockSpec((blkN, 128), lambda i: (i, 0)),
         pl.BlockSpec((128, 64), lambda i: (0, 0)),
         pl.BlockSpec((1, 64), lambda i: (0, 0)),
         pl.BlockSpec((1, 64), lambda i: (0, 0))],
        [pl.BlockSpec((blkN, 80), lambda i: (i, 0)),
         pl.BlockSpec((blkN, 16), lambda i: (i, 0))],
        grid,
    )
    adst1_p = adst1

    sc1 = _make_sc_layer(80, True)
    acc1 = sc1(table1, adst1_p, src, dst)

    table2, asrc2_p, adst2_p = _tc_call(
        _t2_body,
        (acc1, acc1, b1.reshape(1, 64), W2,
         att_src2.reshape(128, 1), att_dst2.reshape(128, 1)),
        (jax.ShapeDtypeStruct((N, 128), _f32),
         jax.ShapeDtypeStruct((NACC, 16), _f32),
         jax.ShapeDtypeStruct((NACC, 16), _f32)),
        [pl.BlockSpec((1, blkN, 80), lambda i: (0, i, 0)),
         pl.BlockSpec((1, blkN, 80), lambda i: (1, i, 0)),
         pl.BlockSpec((1, 64), lambda i: (0, 0)),
         pl.BlockSpec((64, 128), lambda i: (0, 0)),
         pl.BlockSpec((128, 1), lambda i: (0, 0)),
         pl.BlockSpec((128, 1), lambda i: (0, 0))],
        [pl.BlockSpec((blkN, 128), lambda i: (i, 0)),
         pl.BlockSpec((blkN, 16), lambda i: (i, 0)),
         pl.BlockSpec((blkN, 16), lambda i: (i, 0))],
        grid,
    )

    sc2 = _make_sc_layer2()
    accm, accd = sc2(table2, asrc2_p, adst2_p, src, dst)

    out = _tc_call(
        _t3_body,
        (accm, accm, accd, accd, b2.reshape(1, 128)),
        jax.ShapeDtypeStruct((N, 128), _f32),
        [pl.BlockSpec((1, blkN, 128), lambda i: (0, i, 0)),
         pl.BlockSpec((1, blkN, 128), lambda i: (1, i, 0)),
         pl.BlockSpec((1, blkN, 16), lambda i: (0, i, 0)),
         pl.BlockSpec((1, blkN, 16), lambda i: (1, i, 0)),
         pl.BlockSpec((1, 128), lambda i: (0, 0))],
        pl.BlockSpec((blkN, 128), lambda i: (i, 0)),
        grid,
    )
    return out


# speedup vs baseline: 1.2652x; 1.0029x over previous
import functools

import jax
import jax.numpy as jnp
from jax import lax
from jax.experimental import pallas as pl
from jax.experimental.pallas import tpu as pltpu
from jax.experimental.pallas import tpu_sc as plsc

N = 10000
E = 320000
NE = E + N
NC, NS, L = 2, 16, 16
W = NC * NS
BLK = 96
K = 6
NBLK0 = 132
NBLK1 = 84
EP = (NBLK0 + NBLK1) * NS * BLK
NACC = 10240
TROWS = NACC // NS

_f32 = jnp.float32
_i32 = jnp.int32


def _t1_body(x_ref, w1_ref, asf_ref, adf_ref, table_ref, adstv_ref):
    h = jnp.dot(x_ref[...], w1_ref[...], preferred_element_type=_f32)
    hs = _hsel()
    nb = h.shape[0]
    table_ref[:, 0:64] = h
    table_ref[:, 64:72] = jnp.dot(h * asf_ref[...], hs,
                                  preferred_element_type=_f32)
    table_ref[:, 72:80] = jnp.ones((nb, 8), _f32)
    adstv_ref[:, 0:8] = jnp.dot(h * adf_ref[...], hs,
                                preferred_element_type=_f32)
    adstv_ref[:, 8:16] = jnp.zeros((nb, 8), _f32)


def _hsel(transpose=False):
    a = lax.broadcasted_iota(_i32, (64, 8), 0) // 8
    b = lax.broadcasted_iota(_i32, (64, 8), 1)
    m = (a == b).astype(_f32)
    return m.T if transpose else m


def _t2_body(p0_ref, p1_ref, b1_ref, w2_ref, as2_ref, ad2_ref,
             table_ref, asrcv_ref, adstv_ref):
    acc = p0_ref[0] + p1_ref[0]
    recip = 1.0 / acc[:, 72:80]
    out1 = acc[:, 0:64] * jnp.dot(recip, _hsel(True),
                                  preferred_element_type=_f32) + b1_ref[...]
    hact = jnp.where(out1 > 0, out1, jnp.exp(out1) - 1.0)
    h2 = jnp.dot(hact, w2_ref[...], preferred_element_type=_f32)
    table_ref[...] = h2
    nb = h2.shape[0]
    asrcv_ref[:, 0:1] = jnp.dot(h2, as2_ref[...], preferred_element_type=_f32)
    asrcv_ref[:, 1:16] = jnp.zeros((nb, 15), _f32)
    adstv_ref[:, 0:1] = jnp.dot(h2, ad2_ref[...], preferred_element_type=_f32)
    adstv_ref[:, 1:16] = jnp.zeros((nb, 15), _f32)


def _t3_body(m0_ref, m1_ref, d0_ref, d1_ref, b2_ref, out_ref):
    msg = m0_ref[0] + m1_ref[0]
    den = d0_ref[0, :, 0:1] + d1_ref[0, :, 0:1]
    out_ref[...] = msg / den + b2_ref[...]


def _tc_call(body, ins, out_shapes, in_blocks, out_blocks, grid):
    return pl.pallas_call(
        body,
        grid=grid,
        in_specs=in_blocks,
        out_specs=out_blocks,
        out_shape=out_shapes,
    )(*ins)


def _sc_edge_kernel(table, adst, src_p, dst_p, out, acc,
                    sidx0, sidx1, didx0, didx1, rows0, rows1, adb0, adb1,
                    alpha1, gsemT0, gsemT1, gsemA0, gsemA1, ssem0, ssem1,
                    isem0, isem1, *, ncols, nvec, layer1):
    c = lax.axis_index("c")
    s = lax.axis_index("s")
    iota = lax.iota(_i32, L)
    sidx = (sidx0, sidx1)
    didx = (didx0, didx1)
    rows = (rows0, rows1)
    adb = (adb0, adb1)
    gsemT = (gsemT0, gsemT1)
    gsemA = (gsemA0, gsemA1)
    ssem = (ssem0, ssem1)
    isem = (isem0, isem1)

    zero = jnp.zeros((L,), _f32)

    def _zr(r, carry):
        for j in range(nvec):
            rows0[r, pl.ds(16 * j, 16)] = zero
        return carry

    lax.fori_loop(0, BLK, _zr, 0)
    tb = s * TROWS
    nz = TROWS // BLK
    for z in range(nz):
        pltpu.sync_copy(rows0, acc.at[pl.ds(tb + z * BLK, BLK)])
    rem = TROWS - nz * BLK
    if rem:
        pltpu.sync_copy(rows0.at[pl.ds(0, rem)],
                        acc.at[pl.ds(tb + nz * BLK, rem)])
    plsc.subcore_barrier()

    sbrow = jnp.where(c == 0, s * NBLK0, NS * NBLK0 + s * NBLK1)
    nsb_half = jnp.where(c == 0, NBLK0 // K // 2, NBLK1 // K // 2)
    if layer1:
        colpats = [2 * j + lax.shift_right_logical(iota, 3) for j in range(4)]
        colpats.append(iota & 7)

    def _sb_load(sb, q, sem_q):
        pltpu.async_copy(src_p.at[pl.ds(sbrow + sb * K, K)], sidx[q], sem_q)
        pltpu.async_copy(dst_p.at[pl.ds(sbrow + sb * K, K)], didx[q], sem_q)

    def _sb_wait(q):
        pltpu.make_async_copy(src_p.at[pl.ds(0, K)], sidx[q], isem[q]).wait()
        pltpu.make_async_copy(dst_p.at[pl.ds(0, K)], didx[q], isem[q]).wait()

    def _gather_issue(q, k, p):
        pltpu.async_copy(table.at[sidx[q].at[k]], rows[p], gsemT[p])
        pltpu.async_copy(adst.at[didx[q].at[k]], adb[p], gsemA[p])

    def _gather_wait(p):
        pltpu.make_async_copy(table.at[sidx[0].at[0]], rows[p], gsemT[p]).wait()
        pltpu.make_async_copy(adst.at[didx[0].at[0]], adb[p], gsemA[p]).wait()

    def _scatter_issue(q, k, p):
        pltpu.async_copy(rows[p], acc.at[didx[q].at[k]], ssem[p], add=True)

    def _scatter_wait(p):
        pltpu.make_async_copy(rows[p], acc.at[didx[0].at[0]], ssem[p]).wait()

    def _compute(p):
        rp, ap = rows[p], adb[p]

        @plsc.parallel_loop(0, BLK, unroll=4)
        def _edge(e):
            a_s = rp[e, pl.ds(64 if layer1 else 128, 16)]
            a_d = ap[e, pl.ds(0, 16)]
            ssum = a_s + a_d
            alpha = jnp.exp(jnp.where(ssum > 0, ssum, 0.2 * ssum))
            alpha1[pl.ds(e * 16, 16)] = alpha
            if layer1:
                for j in range(nvec - 1):
                    av = plsc.load_gather(alpha1, [e * 16 + colpats[j]])
                    rp[e, pl.ds(16 * j, 16)] = rp[e, pl.ds(16 * j, 16)] * av
                av4 = plsc.load_gather(alpha1, [e * 16 + colpats[nvec - 1]])
                rp[e, pl.ds(16 * (nvec - 1), 16)] = av4
            else:
                av = plsc.load_gather(alpha1, [jnp.full((L,), e * 16, _i32)])
                for j in range(nvec - 1):
                    rp[e, pl.ds(16 * j, 16)] = rp[e, pl.ds(16 * j, 16)] * av
                rp[e, pl.ds(16 * (nvec - 1), 16)] = av

    _sb_load(0, 0, isem[0])
    _sb_wait(0)
    _sb_load(1, 1, isem[1])
    _gather_issue(0, 0, 0)

    def _t_step(t, carry):
        for qq in (0, 1):
            sb = 2 * t + qq
            for k in range(K):
                p = k % 2
                _gather_wait(p)
                if k == 0:
                    if qq == 0:
                        @pl.when(t > 0)
                        def _():
                            _scatter_wait(1 - p)
                            _sb_load(sb + 1, 1 - qq, isem[1 - qq])
                    else:
                        _scatter_wait(1 - p)

                        @pl.when(t < nsb_half - 1)
                        def _():
                            _sb_load(sb + 1, 1 - qq, isem[1 - qq])
                else:
                    _scatter_wait(1 - p)
                if k < K - 1:
                    _gather_issue(qq, k + 1, 1 - p)
                else:
                    if qq == 0:
                        _sb_wait(1 - qq)
                        _gather_issue(1 - qq, 0, 1 - p)
                    else:
                        @pl.when(t < nsb_half - 1)
                        def _():
                            _sb_wait(1 - qq)
                            _gather_issue(1 - qq, 0, 1 - p)
                _compute(p)
                _scatter_issue(qq, k, p)
        return carry

    lax.fori_loop(0, nsb_half, _t_step, 0)
    _scatter_wait(1)
    plsc.subcore_barrier()

    pltpu.sync_copy(acc.at[pl.ds(tb, TROWS)], out.at[c, pl.ds(tb, TROWS)])


def _sc_edge_kernel2(table, asrc, adst, src_p, dst_p, outm, outd, accm, accd,
                     sidx0, sidx1, didx0, didx1, rows0, rows1, asb0, asb1,
                     adb0, adb1, dbuf0, dbuf1, alpha1,
                     gsemT0, gsemT1, gsemS0, gsemS1, gsemA0, gsemA1,
                     ssem0, ssem1, dsem0, dsem1, isem0, isem1):
    c = lax.axis_index("c")
    s = lax.axis_index("s")
    iota = lax.iota(_i32, L)
    sidx = (sidx0, sidx1)
    didx = (didx0, didx1)
    rows = (rows0, rows1)
    asb = (asb0, asb1)
    adb = (adb0, adb1)
    dbuf = (dbuf0, dbuf1)
    gsemT = (gsemT0, gsemT1)
    gsemS = (gsemS0, gsemS1)
    gsemA = (gsemA0, gsemA1)
    ssem = (ssem0, ssem1)
    dsem = (dsem0, dsem1)
    isem = (isem0, isem1)

    zero = jnp.zeros((L,), _f32)

    def _zr(r, carry):
        for j in range(8):
            rows0[r, pl.ds(16 * j, 16)] = zero
        dbuf0[r, pl.ds(0, 16)] = zero
        return carry

    lax.fori_loop(0, BLK, _zr, 0)
    tb = s * TROWS
    nz = TROWS // BLK
    for z in range(nz):
        pltpu.sync_copy(rows0, accm.at[pl.ds(tb + z * BLK, BLK)])
        pltpu.sync_copy(dbuf0, accd.at[pl.ds(tb + z * BLK, BLK)])
    rem = TROWS - nz * BLK
    if rem:
        pltpu.sync_copy(rows0.at[pl.ds(0, rem)],
                        accm.at[pl.ds(tb + nz * BLK, rem)])
        pltpu.sync_copy(dbuf0.at[pl.ds(0, rem)],
                        accd.at[pl.ds(tb + nz * BLK, rem)])
    plsc.subcore_barrier()

    sbrow = jnp.where(c == 0, s * NBLK0, NS * NBLK0 + s * NBLK1)
    nsb_half = jnp.where(c == 0, NBLK0 // K // 2, NBLK1 // K // 2)

    def _sb_load(sb, q, sem_q):
        pltpu.async_copy(src_p.at[pl.ds(sbrow + sb * K, K)], sidx[q], sem_q)
        pltpu.async_copy(dst_p.at[pl.ds(sbrow + sb * K, K)], didx[q], sem_q)

    def _sb_wait(q):
        pltpu.make_async_copy(src_p.at[pl.ds(0, K)], sidx[q], isem[q]).wait()
        pltpu.make_async_copy(dst_p.at[pl.ds(0, K)], didx[q], isem[q]).wait()

    def _gather_issue(q, k, p):
        pltpu.async_copy(table.at[sidx[q].at[k]], rows[p], gsemT[p])
        pltpu.async_copy(asrc.at[sidx[q].at[k]], asb[p], gsemS[p])
        pltpu.async_copy(adst.at[didx[q].at[k]], adb[p], gsemA[p])

    def _gather_wait(p):
        pltpu.make_async_copy(table.at[sidx[0].at[0]], rows[p], gsemT[p]).wait()
        pltpu.make_async_copy(asrc.at[sidx[0].at[0]], asb[p], gsemS[p]).wait()
        pltpu.make_async_copy(adst.at[didx[0].at[0]], adb[p], gsemA[p]).wait()

    def _scatter_issue(q, k, p):
        pltpu.async_copy(rows[p], accm.at[didx[q].at[k]], ssem[p], add=True)
        pltpu.async_copy(dbuf[p], accd.at[didx[q].at[k]], dsem[p], add=True)

    def _scatter_wait(p):
        pltpu.make_async_copy(rows[p], accm.at[didx[0].at[0]], ssem[p]).wait()
        pltpu.make_async_copy(dbuf[p], accd.at[didx[0].at[0]], dsem[p]).wait()

    def _compute(p):
        rp, sp, ap, dp = rows[p], asb[p], adb[p], dbuf[p]

        @plsc.parallel_loop(0, BLK, unroll=4)
        def _edge(e):
            ssum = sp[e, pl.ds(0, 16)] + ap[e, pl.ds(0, 16)]
            alpha = jnp.exp(jnp.where(ssum > 0, ssum, 0.2 * ssum))
            alpha1[pl.ds(e * 16, 16)] = alpha
            av = plsc.load_gather(alpha1, [jnp.full((L,), e * 16, _i32)])
            for j in range(8):
                rp[e, pl.ds(16 * j, 16)] = rp[e, pl.ds(16 * j, 16)] * av
            dp[e, pl.ds(0, 16)] = av

    _sb_load(0, 0, isem[0])
    _sb_wait(0)
    _sb_load(1, 1, isem[1])
    _gather_issue(0, 0, 0)

    def _t_step(t, carry):
        for qq in (0, 1):
            sb = 2 * t + qq
            for k in range(K):
                p = k % 2
                _gather_wait(p)
                if k == 0:
                    if qq == 0:
                        @pl.when(t > 0)
                        def _():
                            _scatter_wait(1 - p)
                            _sb_load(sb + 1, 1 - qq, isem[1 - qq])
                    else:
                        _scatter_wait(1 - p)

                        @pl.when(t < nsb_half - 1)
                        def _():
                            _sb_load(sb + 1, 1 - qq, isem[1 - qq])
                else:
                    _scatter_wait(1 - p)
                if k < K - 1:
                    _gather_issue(qq, k + 1, 1 - p)
                else:
                    if qq == 0:
                        _sb_wait(1 - qq)
                        _gather_issue(1 - qq, 0, 1 - p)
                    else:
                        @pl.when(t < nsb_half - 1)
                        def _():
                            _sb_wait(1 - qq)
                            _gather_issue(1 - qq, 0, 1 - p)
                _compute(p)
                _scatter_issue(qq, k, p)
        return carry

    lax.fori_loop(0, nsb_half, _t_step, 0)
    _scatter_wait(1)
    plsc.subcore_barrier()

    pltpu.sync_copy(accm.at[pl.ds(tb, TROWS)], outm.at[c, pl.ds(tb, TROWS)])
    pltpu.sync_copy(accd.at[pl.ds(tb, TROWS)], outd.at[c, pl.ds(tb, TROWS)])


def _make_sc_layer2():
    mesh = plsc.VectorSubcoreMesh(core_axis_name="c", subcore_axis_name="s")
    return pl.kernel(
        _sc_edge_kernel2,
        out_type=(jax.ShapeDtypeStruct((NC, NACC, 128), _f32),
                  jax.ShapeDtypeStruct((NC, NACC, 16), _f32)),
        mesh=mesh,
        compiler_params=pltpu.CompilerParams(needs_layout_passes=False,
                                             use_tc_tiling_on_sc=False),
        scratch_types=(
            [pltpu.VMEM_SHARED((NACC, 128), _f32),
             pltpu.VMEM_SHARED((NACC, 16), _f32)]
            + [pltpu.VMEM((K, BLK), _i32)] * 4
            + [pltpu.VMEM((BLK, 128), _f32)] * 2
            + [pltpu.VMEM((BLK, 16), _f32)] * 4
            + [pltpu.VMEM((BLK, 16), _f32)] * 2
            + [pltpu.VMEM((BLK * 16,), _f32)]
            + [pltpu.SemaphoreType.DMA] * 12
        ),
    )


def _make_sc_layer(ncols, layer1):
    nvec = ncols // 16
    mesh = plsc.VectorSubcoreMesh(core_axis_name="c", subcore_axis_name="s")
    return pl.kernel(
        functools.partial(_sc_edge_kernel, ncols=ncols, nvec=nvec,
                          layer1=layer1),
        out_type=jax.ShapeDtypeStruct((NC, NACC, ncols), _f32),
        mesh=mesh,
        compiler_params=pltpu.CompilerParams(needs_layout_passes=False,
                                             use_tc_tiling_on_sc=False),
        scratch_types=[
            pltpu.VMEM_SHARED((NACC, ncols), _f32),
            pltpu.VMEM((K, BLK), _i32),
            pltpu.VMEM((K, BLK), _i32),
            pltpu.VMEM((K, BLK), _i32),
            pltpu.VMEM((K, BLK), _i32),
            pltpu.VMEM((BLK, ncols), _f32),
            pltpu.VMEM((BLK, ncols), _f32),
            pltpu.VMEM((BLK, 16), _f32),
            pltpu.VMEM((BLK, 16), _f32),
            pltpu.VMEM((BLK * 16,), _f32),
            pltpu.SemaphoreType.DMA,
            pltpu.SemaphoreType.DMA,
            pltpu.SemaphoreType.DMA,
            pltpu.SemaphoreType.DMA,
            pltpu.SemaphoreType.DMA,
            pltpu.SemaphoreType.DMA,
            pltpu.SemaphoreType.DMA,
            pltpu.SemaphoreType.DMA,
        ],
    )


def kernel(x, edge_index, W1, att_src1, att_dst1, b1, W2, att_src2, att_dst2,
           b2):
    idx = jnp.arange(EP, dtype=_i32)
    s_pad = jnp.pad(edge_index[0].astype(_i32), (0, EP - E))
    d_pad = jnp.pad(edge_index[1].astype(_i32), (0, EP - E))
    src = jnp.where(idx < E, s_pad,
                    jnp.where(idx < NE, idx - E, 0)).reshape(EP // BLK, BLK)
    dst = jnp.where(idx < E, d_pad,
                    jnp.where(idx < NE, idx - E, N)).reshape(EP // BLK, BLK)

    grid = (5,)
    blkN = N // grid[0]

    table1, adst1 = _tc_call(
        _t1_body,
        (x, W1, att_src1.reshape(1, 64), att_dst1.reshape(1, 64)),
        (jax.ShapeDtypeStruct((N, 80), _f32),
         jax.ShapeDtypeStruct((NACC, 16), _f32)),
        [pl.BlockSpec((blkN, 128), lambda i: (i, 0)),
         pl.BlockSpec((128, 64), lambda i: (0, 0)),
         pl.BlockSpec((1, 64), lambda i: (0, 0)),
         pl.BlockSpec((1, 64), lambda i: (0, 0))],
        [pl.BlockSpec((blkN, 80), lambda i: (i, 0)),
         pl.BlockSpec((blkN, 16), lambda i: (i, 0))],
        grid,
    )
    adst1_p = adst1

    sc1 = _make_sc_layer(80, True)
    acc1 = sc1(table1, adst1_p, src, dst)

    table2, asrc2_p, adst2_p = _tc_call(
        _t2_body,
        (acc1, acc1, b1.reshape(1, 64), W2,
         att_src2.reshape(128, 1), att_dst2.reshape(128, 1)),
        (jax.ShapeDtypeStruct((N, 128), _f32),
         jax.ShapeDtypeStruct((NACC, 16), _f32),
         jax.ShapeDtypeStruct((NACC, 16), _f32)),
        [pl.BlockSpec((1, blkN, 80), lambda i: (0, i, 0)),
         pl.BlockSpec((1, blkN, 80), lambda i: (1, i, 0)),
         pl.BlockSpec((1, 64), lambda i: (0, 0)),
         pl.BlockSpec((64, 128), lambda i: (0, 0)),
         pl.BlockSpec((128, 1), lambda i: (0, 0)),
         pl.BlockSpec((128, 1), lambda i: (0, 0))],
        [pl.BlockSpec((blkN, 128), lambda i: (i, 0)),
         pl.BlockSpec((blkN, 16), lambda i: (i, 0)),
         pl.BlockSpec((blkN, 16), lambda i: (i, 0))],
        grid,
    )

    sc2 = _make_sc_layer2()
    accm, accd = sc2(table2, asrc2_p, adst2_p, src, dst)

    out = _tc_call(
        _t3_body,
        (accm, accm, accd, accd, b2.reshape(1, 128)),
        jax.ShapeDtypeStruct((N, 128), _f32),
        [pl.BlockSpec((1, blkN, 128), lambda i: (0, i, 0)),
         pl.BlockSpec((1, blkN, 128), lambda i: (1, i, 0)),
         pl.BlockSpec((1, blkN, 16), lambda i: (0, i, 0)),
         pl.BlockSpec((1, blkN, 16), lambda i: (1, i, 0)),
         pl.BlockSpec((1, 128), lambda i: (0, 0))],
        pl.BlockSpec((blkN, 128), lambda i: (i, 0)),
        grid,
    )
    return out
